# baseline (speedup 1.0000x reference)
import functools

import jax
import jax.numpy as jnp
from jax import lax
from jax.experimental import pallas as pl
from jax.experimental.pallas import tpu as pltpu
from jax.experimental.pallas import tpu_sc as plsc

_BATCH = 4096
_SEQ = 200
_DIM = 64
_NC = 2
_NS = 16
_NW = _NC * _NS
_ROWS_W = _BATCH // _NW
_SPLIT = 104

_GDN = lax.GatherDimensionNumbers(
    offset_dims=(), collapsed_slice_dims=(0,), start_index_map=(0,)
)


def _lane_total(v, lane):
    t = v
    for k in (1, 2, 4, 8):
        perm = (lane + k) & 15
        t = t + lax.gather(
            t, perm[:, None], _GDN, (1,),
            mode=lax.GatherScatterMode.PROMISE_IN_BOUNDS,
        )
    return t[0]


_N = _BATCH * _SEQ
_CHUNKS = 2
_NCHUNK = _N // _CHUNKS
_PER_W = _NCHUNK // _NW
_UNIT = 128
_UNITS = _PER_W // _UNIT


def _emb_body(idx_hbm, table_hbm, out_hbm, idx_v, rows_a, rows_b, sem_a,
              sem_b):
    c = lax.axis_index("c")
    s = lax.axis_index("s")
    wid = s * _NC + c
    base = wid * _PER_W

    pltpu.sync_copy(idx_hbm.at[pl.ds(base, _PER_W)], idx_v)

    zeros16 = jnp.zeros((16,), jnp.float32)
    one = jnp.ones((16,), jnp.int32)
    izero = jnp.zeros((16,), jnp.int32)
    lane = lax.iota(jnp.int32, 16)

    def start_gather(u, buf, sem):
        pltpu.async_copy(
            table_hbm.at[idx_v.at[pl.ds(u * _UNIT, _UNIT)]], buf, sem
        )

    def finish_unit(u, buf, sem):
        pltpu.make_async_copy(
            out_hbm.at[pl.ds(base, _UNIT)], buf, sem
        ).wait()

        m = izero
        for g in range(8):
            m = m + jnp.where(
                idx_v[pl.ds(u * _UNIT + g * 16, 16)] == 0, one, izero
            )

        @pl.when(_lane_total(m, lane) > 0)
        def _fix_unit():
            for g in range(8):
                iv = idx_v[pl.ds(u * _UNIT + g * 16, 16)]
                for l in range(16):
                    @pl.when(iv[l] == 0)
                    def _zero_row(r=g * 16 + l):
                        for j in range(_DIM // 16):
                            buf[r, pl.ds(j * 16, 16)] = zeros16

        pltpu.sync_copy(buf, out_hbm.at[pl.ds(base + u * _UNIT, _UNIT)])

    start_gather(0, rows_a, sem_a)

    def pair(i, carry):
        u0 = i * 2
        start_gather(u0 + 1, rows_b, sem_b)
        finish_unit(u0, rows_a, sem_a)

        @pl.when(u0 + 2 < _UNITS)
        def _next():
            start_gather(u0 + 2, rows_a, sem_a)

        finish_unit(u0 + 1, rows_b, sem_b)
        return carry

    lax.fori_loop(0, _UNITS // 2, pair, 0)


@functools.partial(jax.jit, static_argnames=())
def kernel(event_seq, emb_table):
    idx = event_seq.reshape(_N)
    mesh = plsc.VectorSubcoreMesh(
        core_axis_name="c", subcore_axis_name="s",
        num_cores=_NC, num_subcores=_NS,
    )
    call = pl.kernel(
        _emb_body,
        out_type=jax.ShapeDtypeStruct((_NCHUNK, _DIM), jnp.float32),
        mesh=mesh,
        compiler_params=pltpu.CompilerParams(use_tc_tiling_on_sc=False),
        scratch_types=[
            pltpu.VMEM((_PER_W,), jnp.int32),
            pltpu.VMEM((_UNIT, _DIM), jnp.float32),
            pltpu.VMEM((_UNIT, _DIM), jnp.float32),
            pltpu.SemaphoreType.DMA,
            pltpu.SemaphoreType.DMA,
        ],
    )
    b_chunk = _BATCH // _CHUNKS
    out = jnp.zeros((_BATCH, _SEQ, _DIM), jnp.float32)
    for k in range(_CHUNKS):
        chunk = call(idx[k * _NCHUNK:(k + 1) * _NCHUNK], emb_table)
        out = lax.dynamic_update_slice(
            out, chunk.reshape(b_chunk, _SEQ, _DIM), (k * b_chunk, 0, 0)
        )
    return out


# speedup vs baseline: 1.1287x; 1.1287x over previous
import functools

import jax
import jax.numpy as jnp
from jax import lax
from jax.experimental import pallas as pl
from jax.experimental.pallas import tpu as pltpu
from jax.experimental.pallas import tpu_sc as plsc

_BATCH = 4096
_SEQ = 200
_DIM = 64
_NC = 2
_NS = 16
_NW = _NC * _NS
_ROWS_W = _BATCH // _NW
_SPLIT = 104

_GDN = lax.GatherDimensionNumbers(
    offset_dims=(), collapsed_slice_dims=(0,), start_index_map=(0,)
)


def _lane_total(v, lane):
    t = v
    for k in (1, 2, 4, 8):
        perm = (lane + k) & 15
        t = t + lax.gather(
            t, perm[:, None], _GDN, (1,),
            mode=lax.GatherScatterMode.PROMISE_IN_BOUNDS,
        )
    return t[0]


_N = _BATCH * _SEQ
_PER_W = _N // _NW
_UNIT = 128
_UNITS = _PER_W // _UNIT


def _emb_body(idx_hbm, table_hbm, out_hbm, idx_v, rows_a, rows_b, rows_c,
              sem_a, sem_b, sem_c):
    c = lax.axis_index("c")
    s = lax.axis_index("s")
    wid = s * _NC + c
    base = wid * _PER_W

    pltpu.sync_copy(idx_hbm.at[pl.ds(base, _PER_W)], idx_v)

    zeros16 = jnp.zeros((16,), jnp.float32)
    one = jnp.ones((16,), jnp.int32)
    izero = jnp.zeros((16,), jnp.int32)
    lane = lax.iota(jnp.int32, 16)

    def start_gather(u, buf, sem):
        pltpu.async_copy(
            table_hbm.at[idx_v.at[pl.ds(u * _UNIT, _UNIT)]], buf, sem
        )

    def finish_unit(u, buf, sem):
        pltpu.make_async_copy(
            out_hbm.at[pl.ds(base, _UNIT)], buf, sem
        ).wait()

        m = izero
        for g in range(8):
            m = m + jnp.where(
                idx_v[pl.ds(u * _UNIT + g * 16, 16)] == 0, one, izero
            )

        @pl.when(_lane_total(m, lane) > 0)
        def _fix_unit():
            for g in range(8):
                iv = idx_v[pl.ds(u * _UNIT + g * 16, 16)]
                for l in range(16):
                    @pl.when(iv[l] == 0)
                    def _zero_row(r=g * 16 + l):
                        for j in range(_DIM // 16):
                            buf[r, pl.ds(j * 16, 16)] = zeros16

        pltpu.sync_copy(buf, out_hbm.at[pl.ds(base + u * _UNIT, _UNIT)])

    start_gather(0, rows_a, sem_a)
    start_gather(1, rows_b, sem_b)

    def triple(i, carry):
        u0 = i * 3
        start_gather(u0 + 2, rows_c, sem_c)
        finish_unit(u0, rows_a, sem_a)

        @pl.when(u0 + 3 < _UNITS)
        def _na():
            start_gather(u0 + 3, rows_a, sem_a)

        finish_unit(u0 + 1, rows_b, sem_b)

        @pl.when(u0 + 4 < _UNITS)
        def _nb():
            start_gather(u0 + 4, rows_b, sem_b)

        finish_unit(u0 + 2, rows_c, sem_c)
        return carry

    lax.fori_loop(0, _UNITS // 3, triple, 0)
    finish_unit(_UNITS - 2, rows_a, sem_a)
    finish_unit(_UNITS - 1, rows_b, sem_b)


@functools.partial(jax.jit, static_argnames=())
def kernel(event_seq, emb_table):
    idx = event_seq.reshape(_N)
    mesh = plsc.VectorSubcoreMesh(
        core_axis_name="c", subcore_axis_name="s",
        num_cores=_NC, num_subcores=_NS,
    )
    out = pl.kernel(
        _emb_body,
        out_type=jax.ShapeDtypeStruct((_N, _DIM), jnp.float32),
        mesh=mesh,
        compiler_params=pltpu.CompilerParams(use_tc_tiling_on_sc=False),
        scratch_types=[
            pltpu.VMEM((_PER_W,), jnp.int32),
            pltpu.VMEM((_UNIT, _DIM), jnp.float32),
            pltpu.VMEM((_UNIT, _DIM), jnp.float32),
            pltpu.VMEM((_UNIT, _DIM), jnp.float32),
            pltpu.SemaphoreType.DMA,
            pltpu.SemaphoreType.DMA,
            pltpu.SemaphoreType.DMA,
        ],
    )(idx, emb_table)
    return out.reshape(_BATCH, _SEQ, _DIM)


# speedup vs baseline: 1.1337x; 1.0044x over previous
import functools

import jax
import jax.numpy as jnp
from jax import lax
from jax.experimental import pallas as pl
from jax.experimental.pallas import tpu as pltpu
from jax.experimental.pallas import tpu_sc as plsc

_BATCH = 4096
_SEQ = 200
_DIM = 64
_NC = 2
_NS = 16
_NW = _NC * _NS
_ROWS_W = _BATCH // _NW
_SPLIT = 104

_GDN = lax.GatherDimensionNumbers(
    offset_dims=(), collapsed_slice_dims=(0,), start_index_map=(0,)
)


def _lane_total(v, lane):
    t = v
    for k in (1, 2, 4, 8):
        perm = (lane + k) & 15
        t = t + lax.gather(
            t, perm[:, None], _GDN, (1,),
            mode=lax.GatherScatterMode.PROMISE_IN_BOUNDS,
        )
    return t[0]


_N = _BATCH * _SEQ
_PER_W = _N // _NW
_UNIT = 128
_UNITS = _PER_W // _UNIT


def _emb_body(idx_hbm, table_hbm, out_hbm, idx_v, rows_a, rows_b, rows_c,
              rows_d, sem_a, sem_b, sem_c, sem_d):
    c = lax.axis_index("c")
    s = lax.axis_index("s")
    wid = s * _NC + c
    base = wid * _PER_W

    pltpu.sync_copy(idx_hbm.at[pl.ds(base, _PER_W)], idx_v)

    zeros16 = jnp.zeros((16,), jnp.float32)
    one = jnp.ones((16,), jnp.int32)
    izero = jnp.zeros((16,), jnp.int32)
    lane = lax.iota(jnp.int32, 16)

    def start_gather(u, buf, sem):
        pltpu.async_copy(
            table_hbm.at[idx_v.at[pl.ds(u * _UNIT, _UNIT)]], buf, sem
        )

    def finish_unit(u, buf, sem):
        pltpu.make_async_copy(
            out_hbm.at[pl.ds(base, _UNIT)], buf, sem
        ).wait()

        m = izero
        for g in range(8):
            m = m + jnp.where(
                idx_v[pl.ds(u * _UNIT + g * 16, 16)] == 0, one, izero
            )

        @pl.when(_lane_total(m, lane) > 0)
        def _fix_unit():
            for g in range(8):
                iv = idx_v[pl.ds(u * _UNIT + g * 16, 16)]
                for l in range(16):
                    @pl.when(iv[l] == 0)
                    def _zero_row(r=g * 16 + l):
                        for j in range(_DIM // 16):
                            buf[r, pl.ds(j * 16, 16)] = zeros16

        pltpu.sync_copy(buf, out_hbm.at[pl.ds(base + u * _UNIT, _UNIT)])

    ring = ((rows_a, sem_a), (rows_b, sem_b), (rows_c, sem_c),
            (rows_d, sem_d))
    for p in range(3):
        start_gather(p, *ring[p])

    def quad(i, carry):
        u0 = i * 4
        start_gather(u0 + 3, *ring[3])
        for p in range(4):
            if p:
                nxt = u0 + 3 + p

                @pl.when(nxt < _UNITS)
                def _n(nxt=nxt, p=p):
                    start_gather(nxt, *ring[p - 1])

            finish_unit(u0 + p, *ring[p])
        return carry

    lax.fori_loop(0, _UNITS // 4, quad, 0)


@functools.partial(jax.jit, static_argnames=())
def kernel(event_seq, emb_table):
    idx = event_seq.reshape(_N)
    mesh = plsc.VectorSubcoreMesh(
        core_axis_name="c", subcore_axis_name="s",
        num_cores=_NC, num_subcores=_NS,
    )
    out = pl.kernel(
        _emb_body,
        out_type=jax.ShapeDtypeStruct((_N, _DIM), jnp.float32),
        mesh=mesh,
        compiler_params=pltpu.CompilerParams(use_tc_tiling_on_sc=False),
        scratch_types=[
            pltpu.VMEM((_PER_W,), jnp.int32),
            pltpu.VMEM((_UNIT, _DIM), jnp.float32),
            pltpu.VMEM((_UNIT, _DIM), jnp.float32),
            pltpu.VMEM((_UNIT, _DIM), jnp.float32),
            pltpu.VMEM((_UNIT, _DIM), jnp.float32),
            pltpu.SemaphoreType.DMA,
            pltpu.SemaphoreType.DMA,
            pltpu.SemaphoreType.DMA,
            pltpu.SemaphoreType.DMA,
        ],
    )(idx, emb_table)
    return out.reshape(_BATCH, _SEQ, _DIM)
